# baseline (speedup 1.0000x reference)
import functools

import jax
import jax.numpy as jnp
from jax import lax
from jax.experimental import pallas as pl
from jax.experimental.pallas import tpu as pltpu
from jax.experimental.pallas import tpu_sc as plsc

_RADIUS = 0.2
_NSAMPLE = 32
_LANES = 16
_NWORKERS = 32
_D = 128


def _sc_query_and_group(feat, xs, ys, zs, qx, qy, qz, fps, *, B, N, P, C):
    QPW = (B * P) // _NWORKERS
    NG = N // _LANES
    NS1 = _NSAMPLE + 1
    r2 = _RADIUS * _RADIUS
    NPW = N // 4
    PCH = 256

    mesh = plsc.VectorSubcoreMesh(core_axis_name="c", subcore_axis_name="s",
                                  num_cores=2, num_subcores=16)

    @functools.partial(
        pl.kernel,
        out_type=(jax.ShapeDtypeStruct((B * P * NS1, _D), jnp.float32),
                  jax.ShapeDtypeStruct((B * N, _D), jnp.float32)),
        mesh=mesh,
        compiler_params=pltpu.CompilerParams(needs_layout_passes=False),
        scratch_types=[
            pltpu.VMEM((N,), jnp.float32),
            pltpu.VMEM((N,), jnp.float32),
            pltpu.VMEM((N,), jnp.float32),
            pltpu.VMEM((QPW + _LANES,), jnp.float32),
            pltpu.VMEM((QPW + _LANES,), jnp.float32),
            pltpu.VMEM((QPW + _LANES,), jnp.float32),
            pltpu.VMEM((QPW + _LANES,), jnp.int32),
            pltpu.VMEM((N // 8 + 64, ), jnp.int32),
            pltpu.VMEM((NS1,), jnp.int32),
            pltpu.VMEM((NS1,), jnp.int32),
            pltpu.VMEM((NS1,), jnp.int32),
            pltpu.VMEM((NS1,), jnp.int32),
            pltpu.VMEM((NS1, _D), jnp.float32),
            pltpu.VMEM((NS1, _D), jnp.float32),
            pltpu.VMEM((64 * PCH,), jnp.float32),
            pltpu.VMEM((PCH, _D), jnp.float32),
            pltpu.VMEM((_D,), jnp.int32),
            pltpu.SemaphoreType.DMA,
            pltpu.SemaphoreType.DMA,
            pltpu.SemaphoreType.DMA,
            pltpu.SemaphoreType.DMA,
            pltpu.SemaphoreType.DMA,
        ],
    )
    def k(feat_h, xs_h, ys_h, zs_h, qx_h, qy_h, qz_h, fps_h, out_h, tbl_h,
          xs_v, ys_v, zs_v, qx_v, qy_v, qz_v, fps_v, ibuf,
          ilist_a, ilist_b, olist_a, olist_b, rows_a, rows_b,
          fbuf, trow, tilist,
          gsem_a, gsem_b, ssem_a, ssem_b, bsem):
        w = lax.axis_index("c") * 16 + lax.axis_index("s")
        gq0 = w * QPW
        b = gq0 // P
        base = b * N

        pltpu.sync_copy(xs_h.at[pl.ds(base, N)], xs_v)
        pltpu.sync_copy(ys_h.at[pl.ds(base, N)], ys_v)
        pltpu.sync_copy(zs_h.at[pl.ds(base, N)], zs_v)
        pltpu.sync_copy(qx_h.at[pl.ds(gq0, QPW)], qx_v.at[pl.ds(0, QPW)])
        pltpu.sync_copy(qy_h.at[pl.ds(gq0, QPW)], qy_v.at[pl.ds(0, QPW)])
        pltpu.sync_copy(qz_h.at[pl.ds(gq0, QPW)], qz_v.at[pl.ds(0, QPW)])
        pltpu.sync_copy(fps_h.at[pl.ds(gq0, QPW)], fps_v.at[pl.ds(0, QPW)])

        iota = lax.iota(jnp.int32, _LANES)
        zeros16 = jnp.zeros((_LANES,), jnp.int32)

        n0 = (w % 4) * NPW

        def build_chunk(ch, carry):
            n0c = n0 + ch * PCH
            for c in range(C):
                pltpu.async_copy(
                    feat_h.at[pl.ds((b * C + c) * N + n0c, PCH)],
                    fbuf.at[pl.ds(c * PCH, PCH)], bsem)
            for c in range(C):
                pltpu.make_async_copy(
                    feat_h.at[pl.ds(0, PCH)],
                    fbuf.at[pl.ds(0, PCH)], bsem).wait()

            def tch(c, cc):
                cvec = jnp.full((_LANES,), 6, jnp.int32) + c
                for nb in range(PCH // _LANES):
                    v = fbuf[pl.ds(c * PCH + nb * _LANES, _LANES)]
                    plsc.store_scatter(trow, [iota + nb * _LANES, cvec], v)
                return cc
            lax.fori_loop(0, C, tch, jnp.int32(0))

            for nb in range(PCH // _LANES):
                nvec = iota + nb * _LANES
                px = xs_v[pl.ds(n0c + nb * _LANES, _LANES)]
                py = ys_v[pl.ds(n0c + nb * _LANES, _LANES)]
                pz = zs_v[pl.ds(n0c + nb * _LANES, _LANES)]
                plsc.store_scatter(trow, [nvec, jnp.full((_LANES,), 0, jnp.int32)], px)
                plsc.store_scatter(trow, [nvec, jnp.full((_LANES,), 1, jnp.int32)], py)
                plsc.store_scatter(trow, [nvec, jnp.full((_LANES,), 2, jnp.int32)], pz)

            for half in range(2):
                for j in range(128 // _LANES):
                    tilist[pl.ds(j * _LANES, _LANES)] = (
                        iota + base + n0c + half * 128 + j * _LANES)
                pltpu.async_copy(trow.at[pl.ds(half * 128, 128)],
                                 tbl_h.at[tilist], bsem).wait()
            return carry

        lax.fori_loop(0, NPW // PCH, build_chunk, jnp.int32(0))
        plsc.subcore_barrier()

        def scan_to(q, ilist):
            qxs = qx_v[pl.ds(q, _LANES)][0]
            qys = qy_v[pl.ds(q, _LANES)][0]
            qzs = qz_v[pl.ds(q, _LANES)][0]
            fpsq = fps_v[pl.ds(q, _LANES)][0]
            ibuf[pl.ds(0, _LANES)] = zeros16

            def group(off, cnt_vec):
                dx = xs_v[pl.ds(off, _LANES)] - qxs
                d2 = dx * dx
                dy = ys_v[pl.ds(off, _LANES)] - qys
                d2 = d2 + dy * dy
                dz = zs_v[pl.ds(off, _LANES)] - qzs
                d2 = d2 + dz * dz
                m = d2 < r2
                pref = plsc.cumsum(m.astype(jnp.int32))
                plsc.store_scatter(ibuf, [cnt_vec + pref - 1], iota + off,
                                   mask=m)
                return cnt_vec + plsc.all_reduce_population_count(m)

            GPC = 32
            def super_chunk(sc_i, cnt_vec):
                def run(c):
                    base_off = sc_i * GPC * _LANES

                    @plsc.parallel_loop(0, GPC * _LANES, _LANES, unroll=4,
                                        carry=c)
                    def inner(off, cc):
                        return group(base_off + off, cc)

                    return inner
                return lax.cond(
                    cnt_vec[0] < _NSAMPLE,
                    run,
                    lambda c: c,
                    cnt_vec)

            cnt_vec = lax.fori_loop(0, NG // GPC, super_chunk, zeros16)
            cnt = cnt_vec[0]

            sel_a = ibuf[pl.ds(0, _LANES)]
            sel_b = ibuf[pl.ds(_LANES, _LANES)]
            first = sel_a[0]
            sel_a = jnp.where(iota < cnt, sel_a, first) + base
            sel_b = jnp.where(iota + _LANES < cnt, sel_b, first) + base
            ilist[pl.ds(0, _LANES)] = jnp.where(iota == 0, fpsq + base, 0)
            ilist[pl.ds(1, _LANES)] = sel_a
            ilist[pl.ds(1 + _LANES, _LANES)] = sel_b

        def center_and_olist(q, rows_v, olist):
            qxs = qx_v[pl.ds(q, _LANES)][0]
            qys = qy_v[pl.ds(q, _LANES)][0]
            qzs = qz_v[pl.ds(q, _LANES)][0]
            for chunk in range(3):
                s_idx = iota + chunk * _LANES
                msk = s_idx < NS1
                s_idx = jnp.minimum(s_idx, NS1 - 1)
                for col, qc in ((3, qxs), (4, qys), (5, qzs)):
                    rvec = jnp.full((_LANES,), col - 3, jnp.int32)
                    cvec = jnp.full((_LANES,), col, jnp.int32)
                    v = plsc.load_gather(rows_v, [s_idx, rvec], mask=msk)
                    plsc.store_scatter(rows_v, [s_idx, cvec], v - qc,
                                       mask=msk)
            obase = (gq0 + q) * NS1
            olist[pl.ds(0, _LANES)] = iota + obase
            olist[pl.ds(1, _LANES)] = iota + obase + 1
            olist[pl.ds(1 + _LANES, _LANES)] = iota + obase + 1 + _LANES

        scan_to(jnp.int32(0), ilist_a)

        def pair(i2, carry):
            qa = i2 * 2
            qb = qa + 1

            @pl.when(i2 > 0)
            def _():
                pltpu.make_async_copy(rows_a, out_h.at[olist_a],
                                      ssem_a).wait()
            pltpu.async_copy(tbl_h.at[ilist_a], rows_a, gsem_a)
            scan_to(qb, ilist_b)

            @pl.when(i2 > 0)
            def _():
                pltpu.make_async_copy(rows_b, out_h.at[olist_b],
                                      ssem_b).wait()
            pltpu.async_copy(tbl_h.at[ilist_b], rows_b, gsem_b)

            pltpu.make_async_copy(tbl_h.at[ilist_a], rows_a, gsem_a).wait()
            center_and_olist(qa, rows_a, olist_a)
            pltpu.async_copy(rows_a, out_h.at[olist_a], ssem_a)

            scan_to(jnp.minimum(qa + 2, QPW - 1), ilist_a)

            pltpu.make_async_copy(tbl_h.at[ilist_b], rows_b, gsem_b).wait()
            center_and_olist(qb, rows_b, olist_b)
            pltpu.async_copy(rows_b, out_h.at[olist_b], ssem_b)
            return carry

        lax.fori_loop(0, QPW // 2, pair, jnp.int32(0))
        pltpu.make_async_copy(rows_a, out_h.at[olist_a], ssem_a).wait()
        pltpu.make_async_copy(rows_b, out_h.at[olist_b], ssem_b).wait()

    return k(feat, xs, ys, zs, qx, qy, qz, fps)[0]


def kernel(xyz, new_xyz, features, fps_idx):
    B, N, _ = xyz.shape
    P = new_xyz.shape[1]
    C = features.shape[1]

    feat = features.reshape(-1)
    xs = xyz[..., 0].reshape(-1)
    ys = xyz[..., 1].reshape(-1)
    zs = xyz[..., 2].reshape(-1)
    qx = new_xyz[..., 0].reshape(-1)
    qy = new_xyz[..., 1].reshape(-1)
    qz = new_xyz[..., 2].reshape(-1)
    fps = fps_idx.reshape(-1).astype(jnp.int32)

    out = _sc_query_and_group(feat, xs, ys, zs, qx, qy, qz, fps,
                              B=B, N=N, P=P, C=C)
    out = out.reshape(B, P, _NSAMPLE + 1, _D)[..., :6 + C]
    return jnp.transpose(out, (0, 3, 1, 2))


# speedup vs baseline: 1.0635x; 1.0635x over previous
import functools

import jax
import jax.numpy as jnp
from jax import lax
from jax.experimental import pallas as pl
from jax.experimental.pallas import tpu as pltpu
from jax.experimental.pallas import tpu_sc as plsc

_RADIUS = 0.2
_NSAMPLE = 32
_LANES = 16
_NWORKERS = 32
_D = 128


def _sc_query_and_group(tbl, xyzf, nxyzf, fps, *, B, N, P):
    QPW = (B * P) // _NWORKERS
    NG = N // _LANES
    NS1 = _NSAMPLE + 1
    r2 = _RADIUS * _RADIUS

    mesh = plsc.VectorSubcoreMesh(core_axis_name="c", subcore_axis_name="s",
                                  num_cores=2, num_subcores=16)

    @functools.partial(
        pl.kernel,
        out_type=jax.ShapeDtypeStruct((B * P * NS1, _D), jnp.float32),
        mesh=mesh,
        compiler_params=pltpu.CompilerParams(needs_layout_passes=False),
        scratch_types=[
            pltpu.VMEM((N,), jnp.float32),
            pltpu.VMEM((N,), jnp.float32),
            pltpu.VMEM((N,), jnp.float32),
            pltpu.VMEM((3 * N,), jnp.float32),
            pltpu.VMEM((3 * (QPW + _LANES),), jnp.float32),
            pltpu.VMEM((QPW + _LANES,), jnp.float32),
            pltpu.VMEM((QPW + _LANES,), jnp.float32),
            pltpu.VMEM((QPW + _LANES,), jnp.float32),
            pltpu.VMEM((QPW + _LANES,), jnp.int32),
            pltpu.VMEM((N // 8 + 64, ), jnp.int32),
            pltpu.VMEM((NS1,), jnp.int32),
            pltpu.VMEM((NS1,), jnp.int32),
            pltpu.VMEM((NS1,), jnp.int32),
            pltpu.VMEM((NS1,), jnp.int32),
            pltpu.VMEM((NS1, _D), jnp.float32),
            pltpu.VMEM((NS1, _D), jnp.float32),
            pltpu.SemaphoreType.DMA,
            pltpu.SemaphoreType.DMA,
            pltpu.SemaphoreType.DMA,
            pltpu.SemaphoreType.DMA,
        ],
    )
    def k(tbl_h, xyz_h, nxyz_h, fps_h, out_h,
          xs_v, ys_v, zs_v, xyz3_v, q3_v, qx_v, qy_v, qz_v, fps_v, ibuf,
          ilist_a, ilist_b, olist_a, olist_b, rows_a, rows_b,
          gsem_a, gsem_b, ssem_a, ssem_b):
        w = lax.axis_index("s") * 2 + lax.axis_index("c")
        gq0 = w * QPW
        b = gq0 // P
        base = b * N

        iota = lax.iota(jnp.int32, _LANES)
        zeros16 = jnp.zeros((_LANES,), jnp.int32)

        pltpu.sync_copy(xyz_h.at[pl.ds(base * 3, 3 * N)], xyz3_v)
        pltpu.sync_copy(nxyz_h.at[pl.ds(gq0 * 3, 3 * QPW)],
                        q3_v.at[pl.ds(0, 3 * QPW)])
        pltpu.sync_copy(fps_h.at[pl.ds(gq0, QPW)], fps_v.at[pl.ds(0, QPW)])

        i3 = iota * 3

        @plsc.parallel_loop(0, N, _LANES, unroll=4)
        def _planarize(n0):
            o3 = n0 * 3
            xs_v[pl.ds(n0, _LANES)] = plsc.load_gather(xyz3_v, [i3 + o3])
            ys_v[pl.ds(n0, _LANES)] = plsc.load_gather(xyz3_v, [i3 + o3 + 1])
            zs_v[pl.ds(n0, _LANES)] = plsc.load_gather(xyz3_v, [i3 + o3 + 2])

        for qb in range(QPW // _LANES):
            o3 = qb * _LANES * 3
            qx_v[pl.ds(qb * _LANES, _LANES)] = plsc.load_gather(
                q3_v, [i3 + o3])
            qy_v[pl.ds(qb * _LANES, _LANES)] = plsc.load_gather(
                q3_v, [i3 + o3 + 1])
            qz_v[pl.ds(qb * _LANES, _LANES)] = plsc.load_gather(
                q3_v, [i3 + o3 + 2])

        def scan_to(q, ilist):
            qxs = qx_v[pl.ds(q, _LANES)][0]
            qys = qy_v[pl.ds(q, _LANES)][0]
            qzs = qz_v[pl.ds(q, _LANES)][0]
            fpsq = fps_v[pl.ds(q, _LANES)][0]
            ibuf[pl.ds(0, _LANES)] = zeros16

            def group(off, cnt_vec):
                dx = xs_v[pl.ds(off, _LANES)] - qxs
                d2 = dx * dx
                dy = ys_v[pl.ds(off, _LANES)] - qys
                d2 = d2 + dy * dy
                dz = zs_v[pl.ds(off, _LANES)] - qzs
                d2 = d2 + dz * dz
                m = d2 < r2
                pref = plsc.cumsum(m.astype(jnp.int32))
                plsc.store_scatter(ibuf, [cnt_vec + pref - 1], iota + off,
                                   mask=m)
                return cnt_vec + plsc.all_reduce_population_count(m)

            GPC = 32
            def super_chunk(sc_i, cnt_vec):
                def run(c):
                    base_off = sc_i * GPC * _LANES

                    @plsc.parallel_loop(0, GPC * _LANES, _LANES, unroll=4,
                                        carry=c)
                    def inner(off, cc):
                        return group(base_off + off, cc)

                    return inner
                return lax.cond(
                    cnt_vec[0] < _NSAMPLE,
                    run,
                    lambda c: c,
                    cnt_vec)

            cnt_vec = lax.fori_loop(0, NG // GPC, super_chunk, zeros16)
            cnt = cnt_vec[0]

            sel_a = ibuf[pl.ds(0, _LANES)]
            sel_b = ibuf[pl.ds(_LANES, _LANES)]
            first = sel_a[0]
            sel_a = jnp.where(iota < cnt, sel_a, first) + base
            sel_b = jnp.where(iota + _LANES < cnt, sel_b, first) + base
            ilist[pl.ds(0, _LANES)] = jnp.where(iota == 0, fpsq + base, 0)
            ilist[pl.ds(1, _LANES)] = sel_a
            ilist[pl.ds(1 + _LANES, _LANES)] = sel_b

        def center_and_olist(q, rows_v, olist):
            qxs = qx_v[pl.ds(q, _LANES)][0]
            qys = qy_v[pl.ds(q, _LANES)][0]
            qzs = qz_v[pl.ds(q, _LANES)][0]
            for chunk in range(3):
                s_idx = iota + chunk * _LANES
                msk = s_idx < NS1
                s_idx = jnp.minimum(s_idx, NS1 - 1)
                for col, qc in ((3, qxs), (4, qys), (5, qzs)):
                    cvec = jnp.full((_LANES,), col, jnp.int32)
                    v = plsc.load_gather(rows_v, [s_idx, cvec], mask=msk)
                    plsc.store_scatter(rows_v, [s_idx, cvec], v - qc,
                                       mask=msk)
            obase = (gq0 + q) * NS1
            olist[pl.ds(0, _LANES)] = iota + obase
            olist[pl.ds(1, _LANES)] = iota + obase + 1
            olist[pl.ds(1 + _LANES, _LANES)] = iota + obase + 1 + _LANES

        scan_to(jnp.int32(0), ilist_a)

        def pair(i2, carry):
            qa = i2 * 2
            qb = qa + 1

            @pl.when(i2 > 0)
            def _():
                pltpu.make_async_copy(rows_a, out_h.at[olist_a],
                                      ssem_a).wait()
            pltpu.async_copy(tbl_h.at[ilist_a], rows_a, gsem_a)
            scan_to(qb, ilist_b)

            @pl.when(i2 > 0)
            def _():
                pltpu.make_async_copy(rows_b, out_h.at[olist_b],
                                      ssem_b).wait()
            pltpu.async_copy(tbl_h.at[ilist_b], rows_b, gsem_b)

            pltpu.make_async_copy(tbl_h.at[ilist_a], rows_a, gsem_a).wait()
            center_and_olist(qa, rows_a, olist_a)
            pltpu.async_copy(rows_a, out_h.at[olist_a], ssem_a)

            scan_to(jnp.minimum(qa + 2, QPW - 1), ilist_a)

            pltpu.make_async_copy(tbl_h.at[ilist_b], rows_b, gsem_b).wait()
            center_and_olist(qb, rows_b, olist_b)
            pltpu.async_copy(rows_b, out_h.at[olist_b], ssem_b)
            return carry

        lax.fori_loop(0, QPW // 2, pair, jnp.int32(0))
        pltpu.make_async_copy(rows_a, out_h.at[olist_a], ssem_a).wait()
        pltpu.make_async_copy(rows_b, out_h.at[olist_b], ssem_b).wait()

    return k(tbl, xyzf, nxyzf, fps)


def kernel(xyz, new_xyz, features, fps_idx):
    B, N, _ = xyz.shape
    P = new_xyz.shape[1]
    C = features.shape[1]

    ft = jnp.transpose(features, (0, 2, 1))
    pad = jnp.zeros((B, N, _D - 6 - C), jnp.float32)
    tbl = jnp.concatenate([xyz, xyz, ft, pad], axis=-1).reshape(B * N, _D)
    fps = fps_idx.reshape(-1).astype(jnp.int32)

    out = _sc_query_and_group(tbl, xyz.reshape(-1), new_xyz.reshape(-1), fps,
                              B=B, N=N, P=P)
    out = out.reshape(B, P, _NSAMPLE + 1, _D)[..., :6 + C]
    return jnp.transpose(out, (0, 3, 1, 2))


# speedup vs baseline: 1.1284x; 1.0610x over previous
import functools

import jax
import jax.numpy as jnp
from jax import lax
from jax.experimental import pallas as pl
from jax.experimental.pallas import tpu as pltpu
from jax.experimental.pallas import tpu_sc as plsc

_RADIUS = 0.2
_NSAMPLE = 32
_LANES = 16
_NWORKERS = 32
_D = 128


def _sc_query_and_group(tbl, xs, ys, zs, qx, qy, qz, fps, *, B, N, P):
    QPW = (B * P) // _NWORKERS
    NG = N // _LANES
    NS1 = _NSAMPLE + 1
    r2 = _RADIUS * _RADIUS

    mesh = plsc.VectorSubcoreMesh(core_axis_name="c", subcore_axis_name="s",
                                  num_cores=2, num_subcores=16)

    @functools.partial(
        pl.kernel,
        out_type=jax.ShapeDtypeStruct((B * P * NS1, _D), jnp.float32),
        mesh=mesh,
        compiler_params=pltpu.CompilerParams(needs_layout_passes=False),
        scratch_types=[
            pltpu.VMEM((N,), jnp.float32),
            pltpu.VMEM((N,), jnp.float32),
            pltpu.VMEM((N,), jnp.float32),
            pltpu.VMEM((QPW + _LANES,), jnp.float32),
            pltpu.VMEM((QPW + _LANES,), jnp.float32),
            pltpu.VMEM((QPW + _LANES,), jnp.float32),
            pltpu.VMEM((QPW + _LANES,), jnp.int32),
            pltpu.VMEM((N // 8 + 64, ), jnp.int32),
            pltpu.VMEM((NS1,), jnp.int32),
            pltpu.VMEM((NS1,), jnp.int32),
            pltpu.VMEM((NS1,), jnp.int32),
            pltpu.VMEM((NS1,), jnp.int32),
            pltpu.VMEM((NS1, _D), jnp.float32),
            pltpu.VMEM((NS1, _D), jnp.float32),
            pltpu.SemaphoreType.DMA,
            pltpu.SemaphoreType.DMA,
            pltpu.SemaphoreType.DMA,
            pltpu.SemaphoreType.DMA,
        ],
    )
    def k(tbl_h, xs_h, ys_h, zs_h, qx_h, qy_h, qz_h, fps_h, out_h,
          xs_v, ys_v, zs_v, qx_v, qy_v, qz_v, fps_v, ibuf,
          ilist_a, ilist_b, olist_a, olist_b, rows_a, rows_b,
          gsem_a, gsem_b, ssem_a, ssem_b):
        w = lax.axis_index("s") * 2 + lax.axis_index("c")
        gq0 = w * QPW
        b = gq0 // P
        base = b * N

        pltpu.sync_copy(xs_h.at[pl.ds(base, N)], xs_v)
        pltpu.sync_copy(ys_h.at[pl.ds(base, N)], ys_v)
        pltpu.sync_copy(zs_h.at[pl.ds(base, N)], zs_v)
        pltpu.sync_copy(qx_h.at[pl.ds(gq0, QPW)], qx_v.at[pl.ds(0, QPW)])
        pltpu.sync_copy(qy_h.at[pl.ds(gq0, QPW)], qy_v.at[pl.ds(0, QPW)])
        pltpu.sync_copy(qz_h.at[pl.ds(gq0, QPW)], qz_v.at[pl.ds(0, QPW)])
        pltpu.sync_copy(fps_h.at[pl.ds(gq0, QPW)], fps_v.at[pl.ds(0, QPW)])

        iota = lax.iota(jnp.int32, _LANES)
        zeros16 = jnp.zeros((_LANES,), jnp.int32)

        def scan_to(q, ilist):
            qxs = qx_v[pl.ds(q, _LANES)][0]
            qys = qy_v[pl.ds(q, _LANES)][0]
            qzs = qz_v[pl.ds(q, _LANES)][0]
            fpsq = fps_v[pl.ds(q, _LANES)][0]
            ibuf[pl.ds(0, _LANES)] = zeros16

            def group(off, cnt_vec):
                dx = xs_v[pl.ds(off, _LANES)] - qxs
                d2 = dx * dx
                dy = ys_v[pl.ds(off, _LANES)] - qys
                d2 = d2 + dy * dy
                dz = zs_v[pl.ds(off, _LANES)] - qzs
                d2 = d2 + dz * dz
                m = d2 < r2
                pref = plsc.cumsum(m.astype(jnp.int32))
                plsc.store_scatter(ibuf, [cnt_vec + pref - 1], iota + off,
                                   mask=m)
                return cnt_vec + plsc.all_reduce_population_count(m)

            GPC = 32
            def super_chunk(sc_i, cnt_vec):
                def run(c):
                    base_off = sc_i * GPC * _LANES

                    @plsc.parallel_loop(0, GPC * _LANES, _LANES, unroll=4,
                                        carry=c)
                    def inner(off, cc):
                        return group(base_off + off, cc)

                    return inner
                return lax.cond(
                    cnt_vec[0] < _NSAMPLE,
                    run,
                    lambda c: c,
                    cnt_vec)

            cnt_vec = lax.fori_loop(0, NG // GPC, super_chunk, zeros16)
            cnt = cnt_vec[0]

            sel_a = ibuf[pl.ds(0, _LANES)]
            sel_b = ibuf[pl.ds(_LANES, _LANES)]
            first = sel_a[0]
            sel_a = jnp.where(iota < cnt, sel_a, first) + base
            sel_b = jnp.where(iota + _LANES < cnt, sel_b, first) + base
            ilist[pl.ds(0, _LANES)] = jnp.where(iota == 0, fpsq + base, 0)
            ilist[pl.ds(1, _LANES)] = sel_a
            ilist[pl.ds(1 + _LANES, _LANES)] = sel_b

        def center_and_olist(q, rows_v, olist):
            qxs = qx_v[pl.ds(q, _LANES)][0]
            qys = qy_v[pl.ds(q, _LANES)][0]
            qzs = qz_v[pl.ds(q, _LANES)][0]
            for chunk in range(3):
                s_idx = iota + chunk * _LANES
                msk = s_idx < NS1
                s_idx = jnp.minimum(s_idx, NS1 - 1)
                for col, qc in ((3, qxs), (4, qys), (5, qzs)):
                    cvec = jnp.full((_LANES,), col, jnp.int32)
                    v = plsc.load_gather(rows_v, [s_idx, cvec], mask=msk)
                    plsc.store_scatter(rows_v, [s_idx, cvec], v - qc,
                                       mask=msk)
            obase = (gq0 + q) * NS1
            olist[pl.ds(0, _LANES)] = iota + obase
            olist[pl.ds(1, _LANES)] = iota + obase + 1
            olist[pl.ds(1 + _LANES, _LANES)] = iota + obase + 1 + _LANES

        scan_to(jnp.int32(0), ilist_a)

        def pair(i2, carry):
            qa = i2 * 2
            qb = qa + 1

            @pl.when(i2 > 0)
            def _():
                pltpu.make_async_copy(rows_a, out_h.at[olist_a],
                                      ssem_a).wait()
            pltpu.async_copy(tbl_h.at[ilist_a], rows_a, gsem_a)
            scan_to(qb, ilist_b)

            @pl.when(i2 > 0)
            def _():
                pltpu.make_async_copy(rows_b, out_h.at[olist_b],
                                      ssem_b).wait()
            pltpu.async_copy(tbl_h.at[ilist_b], rows_b, gsem_b)

            pltpu.make_async_copy(tbl_h.at[ilist_a], rows_a, gsem_a).wait()
            center_and_olist(qa, rows_a, olist_a)
            pltpu.async_copy(rows_a, out_h.at[olist_a], ssem_a)

            scan_to(jnp.minimum(qa + 2, QPW - 1), ilist_a)

            pltpu.make_async_copy(tbl_h.at[ilist_b], rows_b, gsem_b).wait()
            center_and_olist(qb, rows_b, olist_b)
            pltpu.async_copy(rows_b, out_h.at[olist_b], ssem_b)
            return carry

        lax.fori_loop(0, QPW // 2, pair, jnp.int32(0))
        pltpu.make_async_copy(rows_a, out_h.at[olist_a], ssem_a).wait()
        pltpu.make_async_copy(rows_b, out_h.at[olist_b], ssem_b).wait()

    return k(tbl, xs, ys, zs, qx, qy, qz, fps)


def kernel(xyz, new_xyz, features, fps_idx):
    B, N, _ = xyz.shape
    P = new_xyz.shape[1]
    C = features.shape[1]

    ft = jnp.transpose(features, (0, 2, 1))
    pad = jnp.zeros((B, N, _D - 6 - C), jnp.float32)
    tbl = jnp.concatenate([xyz, xyz, ft, pad], axis=-1).reshape(B * N, _D)
    xs = xyz[..., 0].reshape(-1)
    ys = xyz[..., 1].reshape(-1)
    zs = xyz[..., 2].reshape(-1)
    qx = new_xyz[..., 0].reshape(-1)
    qy = new_xyz[..., 1].reshape(-1)
    qz = new_xyz[..., 2].reshape(-1)
    fps = fps_idx.reshape(-1).astype(jnp.int32)

    out = _sc_query_and_group(tbl, xs, ys, zs, qx, qy, qz, fps,
                              B=B, N=N, P=P)
    out = out.reshape(B, P, _NSAMPLE + 1, _D)[..., :6 + C]
    return jnp.transpose(out, (0, 3, 1, 2))
